# baseline (speedup 1.0000x reference)
import functools

import jax
import jax.numpy as jnp
from jax import lax
from jax.experimental import pallas as pl
from jax.experimental.pallas import tpu as pltpu
from jax.experimental.pallas import tpu_sc as plsc

_N = 50000
_DIN = 256
_DHID = 512
_DOUT = 256
_T = 4

_NC = 2
_NS = 16
_NW = _NC * _NS
_CPW = 1056
_NR = _NW * _CPW
_NVEC = _CPW // 16
_RB = 96
_NB = _CPW // _RB
_ALIGN = 2048
_M = 43008
_NBLK = 21
_REM = _N - _NR
_MBLK = 384

_mesh = plsc.VectorSubcoreMesh(core_axis_name="c", subcore_axis_name="s")


def _wid():
    return lax.axis_index("s") * _NC + lax.axis_index("c")


def _hist_body(nt_ref, gbase_ref, btp_ref):
    nt = nt_ref[...]
    flat = jnp.concatenate(
        [
            jnp.sum(jnp.where(nt == t, 1, 0).astype(jnp.int32), axis=0,
                    keepdims=True)
            for t in range(_T)
        ],
        axis=0,
    ).astype(jnp.float32)
    nv = _NW * 16
    row = lax.broadcasted_iota(jnp.int32, (nv, nv), 0)
    col = lax.broadcasted_iota(jnp.int32, (nv, nv), 1)
    upper = jnp.where(row < col, 1.0, 0.0)
    ex = jnp.dot(flat, upper, preferred_element_type=jnp.float32)
    per_type = jnp.sum(flat, axis=1, keepdims=True)
    padded = jnp.ceil(per_type / _ALIGN) * _ALIGN
    r4 = lax.broadcasted_iota(jnp.int32, (_T, _T), 0)
    c4 = lax.broadcasted_iota(jnp.int32, (_T, _T), 1)
    lower_inc = jnp.where(r4 >= c4, 1.0, 0.0)
    bounds = jnp.dot(lower_inc, padded,
                     preferred_element_type=jnp.float32)
    starts = bounds - padded
    gbase_ref[...] = (starts + ex).astype(jnp.int32)
    blk_start = (lax.broadcasted_iota(jnp.int32, (1, 256), 1)
                 * _ALIGN).astype(jnp.float32)
    bt = jnp.sum(jnp.where(blk_start >= bounds, 1.0, 0.0), axis=0,
                 keepdims=True)
    btp_ref[...] = jnp.minimum(bt, _T - 1).astype(jnp.int32)


def _count_kernel(nt_by_lane):
    return pl.pallas_call(
        _hist_body,
        out_shape=(
            jax.ShapeDtypeStruct((_T, _NW * 16), jnp.int32),
            jax.ShapeDtypeStruct((1, 256), jnp.int32),
        ),
    )(nt_by_lane)


@functools.partial(
    pl.kernel,
    out_type=(
        jax.ShapeDtypeStruct((_NR,), jnp.int32),
        jax.ShapeDtypeStruct((_M, _DIN), jnp.float32),
    ),
    mesh=_mesh,
    scratch_types=[
        pltpu.VMEM((_CPW,), jnp.int32),
        pltpu.VMEM((_CPW,), jnp.int32),
        pltpu.VMEM((_T, 16), jnp.int32),
        pltpu.VMEM((2, _RB,), jnp.int32),
        pltpu.VMEM((2, _RB, _DIN), jnp.float32),
        pltpu.SemaphoreType.DMA((2,)),
        pltpu.SemaphoreType.DMA((2,)),
    ],
)
def _route_scatter_kernel(nt_hbm, gbase_hbm, x_hbm, dest_hbm, xs_hbm,
                          nt_v, dest_v, gb_v, idx_v, xbuf, semx, sems):
    w = _wid()
    base = w * _CPW
    pltpu.sync_copy(nt_hbm.at[pl.ds(base, _CPW)], nt_v)
    for t in range(_T):
        pltpu.sync_copy(gbase_hbm.at[pl.ds(t * (_NW * 16) + w * 16, 16)],
                        gb_v.at[t])

    offs0 = tuple(gb_v[t] for t in range(_T))
    zeros16 = jnp.zeros((16,), jnp.int32)
    ones16 = jnp.ones((16,), jnp.int32)

    def body(k, offs):
        vt = nt_v[pl.ds(k * 16, 16)]
        dst = zeros16
        new_offs = []
        for t in range(_T):
            m = vt == t
            dst = jnp.where(m, offs[t], dst)
            new_offs.append(offs[t] + jnp.where(m, ones16, zeros16))
        dest_v[pl.ds(k * 16, 16)] = dst
        return tuple(new_offs)

    lax.fori_loop(0, _NVEC, body, offs0)
    pltpu.sync_copy(dest_v, dest_hbm.at[pl.ds(base, _CPW)])

    def start_load(b):
        s = b & 1
        row0 = base + b * _RB
        pltpu.sync_copy(dest_hbm.at[pl.ds(row0, _RB)], idx_v.at[s])
        pltpu.async_copy(x_hbm.at[pl.ds(row0, _RB)], xbuf.at[s], semx.at[s])

    def wait_load(b):
        s = b & 1
        row0 = base + b * _RB
        pltpu.make_async_copy(x_hbm.at[pl.ds(row0, _RB)], xbuf.at[s],
                              semx.at[s]).wait()

    scat = [None, None]
    start_load(0)
    for b in range(_NB):
        s = b & 1
        wait_load(b)
        if b + 1 < _NB:
            if scat[1 - s] is not None:
                scat[1 - s].wait()
            start_load(b + 1)
        scat[s] = pltpu.async_copy(xbuf.at[s], xs_hbm.at[idx_v.at[s]],
                                   sems.at[s])
    scat[0].wait()
    scat[1].wait()


@functools.partial(
    pl.kernel,
    out_type=jax.ShapeDtypeStruct((_NR, _DOUT), jnp.float32),
    mesh=_mesh,
    scratch_types=[
        pltpu.VMEM((2, _RB,), jnp.int32),
        pltpu.VMEM((2, _RB, _DOUT), jnp.float32),
        pltpu.SemaphoreType.DMA((2,)),
    ],
)
def _gather_back_kernel(dest_hbm, os_hbm, out_hbm, idx_v, obuf, sem):
    w = _wid()
    base = w * _CPW

    def start_gather(b):
        s = b & 1
        row0 = base + b * _RB
        pltpu.sync_copy(dest_hbm.at[pl.ds(row0, _RB)], idx_v.at[s])
        pltpu.async_copy(os_hbm.at[idx_v.at[s]], obuf.at[s], sem.at[s])

    def wait_gather(b):
        s = b & 1
        pltpu.make_async_copy(os_hbm.at[idx_v.at[s]], obuf.at[s],
                              sem.at[s]).wait()

    start_gather(0)
    for b in range(_NB):
        s = b & 1
        wait_gather(b)
        if b + 1 < _NB:
            start_gather(b + 1)
        row0 = base + b * _RB
        pltpu.sync_copy(obuf.at[s], out_hbm.at[pl.ds(row0, _RB)])


def _mlp_body(bt_ref, x_ref, w1_ref, b1_ref, w2_ref, b2_ref, o_ref):
    t = bt_ref[pl.program_id(0)]
    h = jnp.maximum(
        jnp.dot(x_ref[...], w1_ref[t], preferred_element_type=jnp.float32)
        + b1_ref[t, 0, :],
        0.0,
    )
    o_ref[...] = (
        jnp.dot(h, w2_ref[t], preferred_element_type=jnp.float32)
        + b2_ref[t, 0, :]
    )


def _mlp_sorted(block_type, xs, W1, b1, W2, b2):
    grid_spec = pltpu.PrefetchScalarGridSpec(
        num_scalar_prefetch=1,
        grid=(_NBLK,),
        in_specs=[
            pl.BlockSpec((_ALIGN, _DIN), lambda i, bt: (i, 0)),
            pl.BlockSpec((_T, _DIN, _DHID), lambda i, bt: (0, 0, 0)),
            pl.BlockSpec((_T, 1, _DHID), lambda i, bt: (0, 0, 0)),
            pl.BlockSpec((_T, _DHID, _DOUT), lambda i, bt: (0, 0, 0)),
            pl.BlockSpec((_T, 1, _DOUT), lambda i, bt: (0, 0, 0)),
        ],
        out_specs=pl.BlockSpec((_ALIGN, _DOUT), lambda i, bt: (i, 0)),
    )
    return pl.pallas_call(
        _mlp_body,
        grid_spec=grid_spec,
        out_shape=jax.ShapeDtypeStruct((_M, _DOUT), jnp.float32),
    )(block_type, xs, W1, b1[:, None, :], W2, b2[:, None, :])


def _masked_body(nt_ref, x_ref, w1_ref, b1_ref, w2_ref, b2_ref, out_ref):
    x = x_ref[...]
    nt = nt_ref[...]
    acc = jnp.zeros_like(out_ref)
    for t in range(_T):
        h = jnp.maximum(
            jnp.dot(x, w1_ref[t], preferred_element_type=jnp.float32)
            + b1_ref[t, 0, :],
            0.0,
        )
        o = jnp.dot(h, w2_ref[t], preferred_element_type=jnp.float32) \
            + b2_ref[t, 0, :]
        acc = acc + jnp.where(nt == t, o, 0.0)
    out_ref[...] = acc


def _masked_tail(nt2, x, W1, b1, W2, b2):
    off = _NR // _MBLK
    return pl.pallas_call(
        _masked_body,
        grid=(pl.cdiv(_REM, _MBLK),),
        in_specs=[
            pl.BlockSpec((_MBLK, 1), lambda i: (i + off, 0)),
            pl.BlockSpec((_MBLK, _DIN), lambda i: (i + off, 0)),
            pl.BlockSpec((_T, _DIN, _DHID), lambda i: (0, 0, 0)),
            pl.BlockSpec((_T, 1, _DHID), lambda i: (0, 0, 0)),
            pl.BlockSpec((_T, _DHID, _DOUT), lambda i: (0, 0, 0)),
            pl.BlockSpec((_T, 1, _DOUT), lambda i: (0, 0, 0)),
        ],
        out_specs=pl.BlockSpec((_MBLK, _DOUT), lambda i: (i, 0)),
        out_shape=jax.ShapeDtypeStruct((_REM, _DOUT), jnp.float32),
    )(nt2, x, W1, b1[:, None, :], W2, b2[:, None, :])


def kernel(raw_features, node_type, W1, b1, W2, b2):
    ntr = node_type[:_NR]
    nt_by_lane = ntr.reshape(_NW, _NVEC, 16).transpose(1, 0, 2).reshape(
        _NVEC, _NW * 16
    )
    gbase, btp = _count_kernel(nt_by_lane)
    block_type = btp[0, :_NBLK]
    dest, xs = _route_scatter_kernel(ntr, gbase.reshape(-1), raw_features)
    os = _mlp_sorted(block_type, xs, W1, b1, W2, b2)
    out_r = _gather_back_kernel(dest, os)
    out_m = _masked_tail(node_type.reshape(_N, 1), raw_features,
                         W1, b1, W2, b2)
    return jnp.concatenate([out_r, out_m], axis=0)


# speedup vs baseline: 1.2435x; 1.2435x over previous
import functools

import jax
import jax.numpy as jnp
from jax import lax
from jax.experimental import pallas as pl
from jax.experimental.pallas import tpu as pltpu
from jax.experimental.pallas import tpu_sc as plsc

_N = 50000
_DIN = 256
_DHID = 512
_DOUT = 256
_T = 4

_NC = 2
_NS = 16
_NW = _NC * _NS
_CPW = 1568
_NPAD = _NW * _CPW
_NVEC = _CPW // 16
_RB = 112
_NB = _CPW // _RB
_ALIGN = 2048
_M = 59392
_NBLK = 29
_LAST_FULL_B = 12
_LAST_TAIL = 48

_mesh = plsc.VectorSubcoreMesh(core_axis_name="c", subcore_axis_name="s")


def _wid():
    return lax.axis_index("s") * _NC + lax.axis_index("c")


def _hist_body(nt_ref, gbase_ref, btp_ref):
    nt = nt_ref[...]
    flat = jnp.concatenate(
        [
            jnp.sum(jnp.where(nt == t, 1, 0).astype(jnp.int32), axis=0,
                    keepdims=True)
            for t in range(_T)
        ],
        axis=0,
    ).astype(jnp.float32)
    nv = _NW * 16
    row = lax.broadcasted_iota(jnp.int32, (nv, nv), 0)
    col = lax.broadcasted_iota(jnp.int32, (nv, nv), 1)
    upper = jnp.where(row < col, 1.0, 0.0)
    ex = jnp.dot(flat, upper, preferred_element_type=jnp.float32)
    per_type = jnp.sum(flat, axis=1, keepdims=True)
    padded = jnp.ceil(per_type / _ALIGN) * _ALIGN
    r4 = lax.broadcasted_iota(jnp.int32, (_T, _T), 0)
    c4 = lax.broadcasted_iota(jnp.int32, (_T, _T), 1)
    lower_inc = jnp.where(r4 >= c4, 1.0, 0.0)
    bounds = jnp.dot(lower_inc, padded,
                     preferred_element_type=jnp.float32)
    starts = bounds - padded
    gbase_ref[...] = (starts + ex).astype(jnp.int32)
    blk_start = (lax.broadcasted_iota(jnp.int32, (1, 256), 1) * _ALIGN).astype(jnp.float32)
    bt = jnp.sum(jnp.where(blk_start >= bounds, 1.0, 0.0), axis=0,
                 keepdims=True)
    btp_ref[...] = jnp.minimum(bt, _T - 1).astype(jnp.int32)


def _count_kernel(nt_by_lane):
    return pl.pallas_call(
        _hist_body,
        out_shape=(
            jax.ShapeDtypeStruct((_T, _NW * 16), jnp.int32),
            jax.ShapeDtypeStruct((1, 256), jnp.int32),
        ),
    )(nt_by_lane)


@functools.partial(
    pl.kernel,
    out_type=(
        jax.ShapeDtypeStruct((_NPAD,), jnp.int32),
        jax.ShapeDtypeStruct((_M, _DIN), jnp.float32),
    ),
    mesh=_mesh,
    scratch_types=[
        pltpu.VMEM((_CPW,), jnp.int32),
        pltpu.VMEM((_CPW,), jnp.int32),
        pltpu.VMEM((_T, 16), jnp.int32),
        pltpu.VMEM((2, _RB,), jnp.int32),
        pltpu.VMEM((2, _RB, _DIN), jnp.float32),
        pltpu.SemaphoreType.DMA((2,)),
        pltpu.SemaphoreType.DMA((2,)),
    ],
)
def _route_scatter_kernel(nt_hbm, gbase_hbm, x_hbm, dest_hbm, xs_hbm,
                          nt_v, dest_v, gb_v, idx_v, xbuf, semx, sems):
    w = _wid()
    base = w * _CPW
    pltpu.sync_copy(nt_hbm.at[pl.ds(base, _CPW)], nt_v)
    for t in range(_T):
        pltpu.sync_copy(gbase_hbm.at[pl.ds(t * (_NW * 16) + w * 16, 16)],
                        gb_v.at[t])

    offs0 = tuple(gb_v[t] for t in range(_T))
    zeros16 = jnp.zeros((16,), jnp.int32)
    ones16 = jnp.ones((16,), jnp.int32)

    def body(k, offs):
        vt = nt_v[pl.ds(k * 16, 16)]
        dst = zeros16
        new_offs = []
        for t in range(_T):
            m = vt == t
            dst = jnp.where(m, offs[t], dst)
            new_offs.append(offs[t] + jnp.where(m, ones16, zeros16))
        dest_v[pl.ds(k * 16, 16)] = dst
        return tuple(new_offs)

    lax.fori_loop(0, _NVEC, body, offs0)
    pltpu.sync_copy(dest_v, dest_hbm.at[pl.ds(base, _CPW)])

    is_last = w == _NW - 1

    def start_load(b):
        s = b & 1
        row0 = base + b * _RB
        pltpu.sync_copy(dest_hbm.at[pl.ds(row0, _RB)], idx_v.at[s])
        if b < _LAST_FULL_B:
            pltpu.async_copy(x_hbm.at[pl.ds(row0, _RB)], xbuf.at[s],
                             semx.at[s])
            return

        @pl.when(jnp.logical_not(is_last))
        def _():
            pltpu.async_copy(x_hbm.at[pl.ds(row0, _RB)], xbuf.at[s],
                             semx.at[s])

        if b == _LAST_FULL_B:
            @pl.when(is_last)
            def _():
                pltpu.async_copy(x_hbm.at[pl.ds(row0, _LAST_TAIL)],
                                 xbuf.at[s, pl.ds(0, _LAST_TAIL)],
                                 semx.at[s])

    def wait_load(b):
        s = b & 1
        row0 = base + b * _RB
        if b < _LAST_FULL_B:
            pltpu.make_async_copy(x_hbm.at[pl.ds(row0, _RB)], xbuf.at[s],
                                  semx.at[s]).wait()
            return

        @pl.when(jnp.logical_not(is_last))
        def _():
            pltpu.make_async_copy(x_hbm.at[pl.ds(row0, _RB)], xbuf.at[s],
                                  semx.at[s]).wait()

        if b == _LAST_FULL_B:
            @pl.when(is_last)
            def _():
                pltpu.make_async_copy(x_hbm.at[pl.ds(row0, _LAST_TAIL)],
                                      xbuf.at[s, pl.ds(0, _LAST_TAIL)],
                                      semx.at[s]).wait()

    scat = [None, None]
    start_load(0)
    for b in range(_NB):
        s = b & 1
        wait_load(b)
        if b + 1 < _NB:
            if scat[1 - s] is not None:
                scat[1 - s].wait()
            start_load(b + 1)
        scat[s] = pltpu.async_copy(xbuf.at[s], xs_hbm.at[idx_v.at[s]],
                                   sems.at[s])
    scat[0].wait()
    scat[1].wait()


@functools.partial(
    pl.kernel,
    out_type=jax.ShapeDtypeStruct((_N, _DOUT), jnp.float32),
    mesh=_mesh,
    scratch_types=[
        pltpu.VMEM((2, _RB,), jnp.int32),
        pltpu.VMEM((2, _RB, _DOUT), jnp.float32),
        pltpu.SemaphoreType.DMA((2,)),
    ],
)
def _gather_back_kernel(dest_hbm, os_hbm, out_hbm, idx_v, obuf, sem):
    w = _wid()
    base = w * _CPW
    is_last = w == _NW - 1

    def start_gather(b):
        s = b & 1
        row0 = base + b * _RB
        pltpu.sync_copy(dest_hbm.at[pl.ds(row0, _RB)], idx_v.at[s])
        pltpu.async_copy(os_hbm.at[idx_v.at[s]], obuf.at[s], sem.at[s])

    def wait_gather(b):
        s = b & 1
        pltpu.make_async_copy(os_hbm.at[idx_v.at[s]], obuf.at[s],
                              sem.at[s]).wait()

    start_gather(0)
    for b in range(_NB):
        s = b & 1
        wait_gather(b)
        if b + 1 < _NB:
            start_gather(b + 1)
        row0 = base + b * _RB
        if b < _LAST_FULL_B:
            pltpu.sync_copy(obuf.at[s], out_hbm.at[pl.ds(row0, _RB)])
        elif b == _LAST_FULL_B:
            @pl.when(jnp.logical_not(is_last))
            def _():
                pltpu.sync_copy(obuf.at[s], out_hbm.at[pl.ds(row0, _RB)])

            @pl.when(is_last)
            def _():
                pltpu.sync_copy(obuf.at[s, pl.ds(0, _LAST_TAIL)],
                                out_hbm.at[pl.ds(row0, _LAST_TAIL)])
        else:
            @pl.when(jnp.logical_not(is_last))
            def _():
                pltpu.sync_copy(obuf.at[s], out_hbm.at[pl.ds(row0, _RB)])


def _mlp_body(bt_ref, x_ref, w1_ref, b1_ref, w2_ref, b2_ref, o_ref):
    t = bt_ref[pl.program_id(0)]
    h = jnp.maximum(
        jnp.dot(x_ref[...], w1_ref[t], preferred_element_type=jnp.float32)
        + b1_ref[t, 0, :],
        0.0,
    )
    o_ref[...] = (
        jnp.dot(h, w2_ref[t], preferred_element_type=jnp.float32)
        + b2_ref[t, 0, :]
    )


def _mlp_sorted(block_type, xs, W1, b1, W2, b2):
    grid_spec = pltpu.PrefetchScalarGridSpec(
        num_scalar_prefetch=1,
        grid=(_NBLK,),
        in_specs=[
            pl.BlockSpec((_ALIGN, _DIN), lambda i, bt: (i, 0)),
            pl.BlockSpec((_T, _DIN, _DHID), lambda i, bt: (0, 0, 0)),
            pl.BlockSpec((_T, 1, _DHID), lambda i, bt: (0, 0, 0)),
            pl.BlockSpec((_T, _DHID, _DOUT), lambda i, bt: (0, 0, 0)),
            pl.BlockSpec((_T, 1, _DOUT), lambda i, bt: (0, 0, 0)),
        ],
        out_specs=pl.BlockSpec((_ALIGN, _DOUT), lambda i, bt: (i, 0)),
    )
    return pl.pallas_call(
        _mlp_body,
        grid_spec=grid_spec,
        out_shape=jax.ShapeDtypeStruct((_M, _DOUT), jnp.float32),
    )(block_type, xs, W1, b1[:, None, :], W2, b2[:, None, :])


def kernel(raw_features, node_type, W1, b1, W2, b2):
    ntp = jnp.concatenate(
        [node_type, jnp.full((_NPAD - _N,), _T - 1, jnp.int32)]
    )
    nt_by_lane = ntp.reshape(_NW, _NVEC, 16).transpose(1, 0, 2).reshape(
        _NVEC, _NW * 16
    )
    gbase, btp = _count_kernel(nt_by_lane)
    block_type = btp[0, :_NBLK]
    dest, xs = _route_scatter_kernel(ntp, gbase.reshape(-1), raw_features)
    os = _mlp_sorted(block_type, xs, W1, b1, W2, b2)
    return _gather_back_kernel(dest, os)
